# baseline (speedup 1.0000x reference)
import functools

import jax
import jax.numpy as jnp
from jax.experimental import pallas as pl
from jax.experimental.pallas import tpu as pltpu


def _gap_matmul_kernel(x_ref, s_ref, o_ref, *, inv_hw):
    acc = jax.lax.dot_general(
        x_ref[...], s_ref[...],
        dimension_numbers=(((1,), (0,)), ((), ())),
        preferred_element_type=jnp.float32)
    o_ref[...] = (acc * inv_hw).astype(o_ref.dtype)


def _segment_matrix(k, lanes=128):
    seg_of = jnp.arange(k * lanes, dtype=jnp.int32) // k
    return (seg_of[:, None] == jnp.arange(lanes, dtype=jnp.int32)[None, :]
            ).astype(jnp.float32)


def _global_avg_pool2d(x_nchw, *, m_blk=256):
    N, C, H, W = x_nchw.shape
    K = H * W
    M = N * C
    LANES = 128
    L = K * LANES
    R = M // LANES

    x2d = x_nchw.reshape(R, L)
    s = _segment_matrix(K, LANES)
    inv_hw = 1.0 / float(K)

    grid = (R // m_blk,)
    out2d = pl.pallas_call(
        functools.partial(_gap_matmul_kernel, inv_hw=inv_hw),
        out_shape=jax.ShapeDtypeStruct((R, LANES), x_nchw.dtype),
        grid_spec=pltpu.PrefetchScalarGridSpec(
            num_scalar_prefetch=0,
            grid=grid,
            in_specs=[
                pl.BlockSpec((m_blk, L), lambda i: (i, 0)),
                pl.BlockSpec((L, LANES), lambda i: (0, 0)),
            ],
            out_specs=pl.BlockSpec((m_blk, LANES), lambda i: (i, 0)),
        ),
        compiler_params=pltpu.CompilerParams(
            dimension_semantics=("parallel",)),
    )(x2d, s)

    return out2d.reshape(N, C, 1, 1)


def kernel(x_nchw):
    return _global_avg_pool2d(x_nchw)


# speedup vs baseline: 32.3308x; 32.3308x over previous
import functools

import jax
import jax.numpy as jnp
from jax.experimental import pallas as pl
from jax.experimental.pallas import tpu as pltpu


def _plane_sum_kernel(x_ref, o_ref, *, inv_hw):
    s = jnp.sum(x_ref[...].astype(jnp.float32), axis=0)
    o_ref[...] = (s * inv_hw).astype(o_ref.dtype)


def _global_avg_pool2d(x_nchw, *, n_blk=16):
    N, C, H, W = x_nchw.shape
    HW = H * W

    planes = jnp.transpose(x_nchw, (2, 3, 0, 1)).reshape(HW, N, C)
    inv_hw = 1.0 / float(HW)

    out2d = pl.pallas_call(
        functools.partial(_plane_sum_kernel, inv_hw=inv_hw),
        out_shape=jax.ShapeDtypeStruct((N, C), x_nchw.dtype),
        grid_spec=pltpu.PrefetchScalarGridSpec(
            num_scalar_prefetch=0,
            grid=(N // n_blk,),
            in_specs=[pl.BlockSpec((HW, n_blk, C), lambda i: (0, i, 0))],
            out_specs=pl.BlockSpec((n_blk, C), lambda i: (i, 0)),
        ),
        compiler_params=pltpu.CompilerParams(
            dimension_semantics=("parallel",)),
    )(planes)

    return out2d.reshape(N, C, 1, 1)


def kernel(x_nchw):
    return _global_avg_pool2d(x_nchw)
